# baseline (speedup 1.0000x reference)
import functools

import jax
import jax.numpy as jnp
from jax import lax
from jax.experimental import pallas as pl
from jax.experimental.pallas import tpu as pltpu
from jax.experimental.pallas import tpu_sc as plsc

_B = 4096
_EMB = 64
_HID = (512, 256, 128)
_TEMP = 0.1
_BN_INV = float(1.0 / (1.0 + 1e-5) ** 0.5)
_NF = 8


def _sc_gather(mega_idx, mega_tab):
    info = plsc.get_sparse_core_info()
    nc, ns = info.num_cores, info.num_subcores
    nw = nc * ns
    n = _NF * _B
    bpw = n // nw

    mesh = plsc.VectorSubcoreMesh(core_axis_name="c", subcore_axis_name="s")

    @functools.partial(
        pl.kernel,
        mesh=mesh,
        out_type=jax.ShapeDtypeStruct((n, _EMB), jnp.float32),
        scratch_types=[
            pltpu.VMEM((bpw,), jnp.int32),
            pltpu.VMEM((bpw, _EMB), jnp.float32),
            pltpu.SemaphoreType.DMA,
        ],
        compiler_params=pltpu.CompilerParams(use_tc_tiling_on_sc=False),
    )
    def gather_kernel(idx_ref, tab_ref, out_ref, idx_v, rows_v, sem):
        wid = lax.axis_index("s") * nc + lax.axis_index("c")
        base = wid * bpw
        pltpu.sync_copy(idx_ref.at[pl.ds(base, bpw)], idx_v)
        pltpu.async_copy(tab_ref.at[idx_v], rows_v, sem).wait()
        pltpu.sync_copy(rows_v, out_ref.at[pl.ds(base, bpw)])

    return gather_kernel(mega_idx, mega_tab)


_T_BLK = 1024


def _tower_block(e_refs, w_refs):
    w0, b0, g0, bt0, w1, b1, g1, bt1, w2, b2, g2, bt2 = w_refs
    bf = jnp.bfloat16
    x = None
    for f in range(4):
        part = jnp.dot(
            e_refs[f][0].astype(bf),
            w0[f * _EMB:(f + 1) * _EMB, :].astype(bf),
            preferred_element_type=jnp.float32,
        )
        x = part if x is None else x + part
    x = jnp.maximum(x + b0[...], 0.0)
    x = (g0[...] * _BN_INV) * x + bt0[...]
    x = jnp.dot(x.astype(bf), w1[...].astype(bf),
                preferred_element_type=jnp.float32)
    x = jnp.maximum(x + b1[...], 0.0)
    x = (g1[...] * _BN_INV) * x + bt1[...]
    x = jnp.dot(x.astype(bf), w2[...].astype(bf),
                preferred_element_type=jnp.float32)
    x = jnp.maximum(x + b2[...], 0.0)
    x = (g2[...] * _BN_INV) * x + bt2[...]
    nrm = jnp.sqrt(jnp.sum(x * x, axis=-1, keepdims=True))
    return x / jnp.maximum(nrm, 1e-12)


def _towers_kernel(*refs):
    eu = refs[0:4]
    ei = refs[4:8]
    wu = refs[8:20]
    wi = refs[20:32]
    ue_ref, ie_ref = refs[32], refs[33]
    ue_ref[...] = _tower_block(eu, wu)
    ie_ref[...] = _tower_block(ei, wi)


def _towers_tc(gathered, wu, wi):
    nblk = _B // _T_BLK

    def _e_spec(f):
        return pl.BlockSpec((1, _T_BLK, _EMB), lambda i, _f=f: (_f, i, 0))

    def _full(a):
        nd = a.ndim
        return pl.BlockSpec(a.shape, lambda i, _n=nd: (0,) * _n)

    in_specs = (
        [_e_spec(f) for f in range(8)]
        + [_full(a) for a in wu]
        + [_full(a) for a in wi]
    )
    out_spec = pl.BlockSpec((_T_BLK, _HID[-1]), lambda i: (i, 0))
    out_shape = (
        jax.ShapeDtypeStruct((_B, _HID[-1]), jnp.float32),
        jax.ShapeDtypeStruct((_B, _HID[-1]), jnp.float32),
    )
    return pl.pallas_call(
        _towers_kernel,
        grid=(nblk,),
        in_specs=in_specs,
        out_specs=(out_spec, out_spec),
        out_shape=out_shape,
    )(*([gathered] * 8), *wu, *wi)


_L_BLK = 512


def _logits_kernel(ue_ref, ie_ref, out_ref):
    out_ref[...] = lax.dot_general(
        ue_ref[...].astype(jnp.bfloat16),
        ie_ref[...].astype(jnp.bfloat16),
        (((1,), (1,)), ((), ())),
        preferred_element_type=jnp.float32,
    ) * (1.0 / _TEMP)


def _logits_tc(ue, ie):
    nblk = _B // _L_BLK
    return pl.pallas_call(
        _logits_kernel,
        grid=(nblk,),
        in_specs=[
            pl.BlockSpec((_L_BLK, _HID[-1]), lambda i: (i, 0)),
            pl.BlockSpec((_B, _HID[-1]), lambda i: (0, 0)),
        ],
        out_specs=pl.BlockSpec((_L_BLK, _B), lambda i: (i, 0)),
        out_shape=jax.ShapeDtypeStruct((_B, _B), jnp.float32),
    )(ue, ie)


def kernel(
    user_id, emb_user_id,
    user_age, emb_user_age,
    user_gender, emb_user_gender,
    user_region, emb_user_region,
    item_id, emb_item_id,
    item_category, emb_item_category,
    item_brand, emb_item_brand,
    item_price_bucket, emb_item_price_bucket,
    u_W0, u_b0, u_g0, u_beta0,
    u_W1, u_b1, u_g1, u_beta1,
    u_W2, u_b2, u_g2, u_beta2,
    i_W0, i_b0, i_g0, i_beta0,
    i_W1, i_b1, i_g1, i_beta1,
    i_W2, i_b2, i_g2, i_beta2,
):
    idxs = (user_id, user_age, user_gender, user_region,
            item_id, item_category, item_brand, item_price_bucket)
    tabs = (emb_user_id, emb_user_age, emb_user_gender, emb_user_region,
            emb_item_id, emb_item_category, emb_item_brand,
            emb_item_price_bucket)
    off = 0
    mega_idx = []
    for ix, t in zip(idxs, tabs):
        mega_idx.append(jnp.asarray(ix, jnp.int32) + off)
        off += t.shape[0]
    mega_idx = jnp.concatenate(mega_idx)
    mega_tab = jnp.concatenate(tabs, axis=0)

    gathered = _sc_gather(mega_idx, mega_tab).reshape(_NF, _B, _EMB)

    def _prep(b, g, bt):
        return (b.reshape(1, -1), g.reshape(1, -1), bt.reshape(1, -1))

    wu = (u_W0, *_prep(u_b0, u_g0, u_beta0),
          u_W1, *_prep(u_b1, u_g1, u_beta1),
          u_W2, *_prep(u_b2, u_g2, u_beta2))
    wi = (i_W0, *_prep(i_b0, i_g0, i_beta0),
          i_W1, *_prep(i_b1, i_g1, i_beta1),
          i_W2, *_prep(i_b2, i_g2, i_beta2))

    ue, ie = _towers_tc(gathered, wu, wi)
    return _logits_tc(ue, ie)


# speedup vs baseline: 3.3041x; 3.3041x over previous
import functools

import jax
import jax.numpy as jnp
from jax import lax
from jax.experimental import pallas as pl
from jax.experimental.pallas import tpu as pltpu
from jax.experimental.pallas import tpu_sc as plsc

_B = 4096
_EMB = 64
_HID = (512, 256, 128)
_TEMP = 0.1
_BN_INV = float(1.0 / (1.0 + 1e-5) ** 0.5)
_SEG = 128
_VMAX = 100096


def _sc_gather_t(idxs, tabs_t, tails_t):
    info = plsc.get_sparse_core_info()
    nc, ns, nl = info.num_cores, info.num_subcores, info.num_lanes
    nw = nc * ns
    rows_per_w = _EMB // nw

    mesh = plsc.VectorSubcoreMesh(core_axis_name="c", subcore_axis_name="s")

    @functools.partial(
        pl.kernel,
        mesh=mesh,
        out_type=tuple(
            jax.ShapeDtypeStruct((_EMB, _B), jnp.float32) for _ in range(8)
        ),
        scratch_types=[
            pltpu.VMEM((_VMAX,), jnp.float32),
            pltpu.VMEM((_B,), jnp.int32),
            pltpu.VMEM((_B,), jnp.float32),
            pltpu.SemaphoreType.DMA,
            pltpu.SemaphoreType.DMA,
        ],
        compiler_params=pltpu.CompilerParams(needs_layout_passes=False),
    )
    def gather_kernel(*refs):
        idx_refs = refs[0:8]
        tab_refs = refs[8:16]
        tail_refs = refs[16:24]
        out_refs = refs[24:32]
        row_v, idx_v, out_v = refs[32], refs[33], refs[34]
        sem_r, sem_w = refs[35], refs[36]
        wid = lax.axis_index("s") * nc + lax.axis_index("c")

        for f in range(8):
            v = tabs_t[f].shape[1]
            nfull = v // _SEG
            tail = v - nfull * _SEG
            pltpu.sync_copy(idx_refs[f], idx_v)
            for r in range(rows_per_w):
                c = wid * rows_per_w + r

                if nfull > 0:
                    def fire(s, carry):
                        pltpu.async_copy(
                            tab_refs[f].at[c, pl.ds(s * _SEG, _SEG)],
                            row_v.at[pl.ds(s * _SEG, _SEG)],
                            sem_r,
                        )
                        return carry
                    lax.fori_loop(0, nfull, fire, 0, unroll=4)
                if tail:
                    pltpu.async_copy(
                        tail_refs[f].at[c, :],
                        row_v.at[pl.ds(nfull * _SEG, _SEG)],
                        sem_r,
                    )
                if nfull > 0:
                    pltpu.make_async_copy(
                        tab_refs[f].at[c, pl.ds(0, nfull * _SEG)],
                        row_v.at[pl.ds(0, nfull * _SEG)],
                        sem_r,
                    ).wait()
                if tail:
                    pltpu.make_async_copy(
                        tail_refs[f].at[c, :],
                        row_v.at[pl.ds(nfull * _SEG, _SEG)],
                        sem_r,
                    ).wait()

                def gat(k, carry):
                    iv = idx_v[pl.ds(k * nl, nl)]
                    out_v[pl.ds(k * nl, nl)] = plsc.load_gather(row_v, [iv])
                    return carry
                lax.fori_loop(0, _B // nl, gat, 0, unroll=4)

                def put(s, carry):
                    pltpu.async_copy(
                        out_v.at[pl.ds(s * _SEG, _SEG)],
                        out_refs[f].at[c, pl.ds(s * _SEG, _SEG)],
                        sem_w,
                    )
                    return carry
                lax.fori_loop(0, _B // _SEG, put, 0, unroll=4)
                pltpu.make_async_copy(
                    out_v,
                    out_refs[f].at[c, pl.ds(0, _B)],
                    sem_w,
                ).wait()

    return gather_kernel(*idxs, *tabs_t, *tails_t)


_T_BLK = 1024


def _tower_block(e_refs, w_refs):
    w0, b0, g0, bt0, w1, b1, g1, bt1, w2, b2, g2, bt2 = w_refs
    bf = jnp.bfloat16
    x = None
    for f in range(4):
        part = jnp.dot(
            w0[:, f * _EMB:(f + 1) * _EMB].astype(bf),
            e_refs[f][...].astype(bf),
            preferred_element_type=jnp.float32,
        )
        x = part if x is None else x + part
    x = jnp.maximum(x + b0[...], 0.0)
    x = (g0[...] * _BN_INV) * x + bt0[...]
    x = jnp.dot(w1[...].astype(bf), x.astype(bf),
                preferred_element_type=jnp.float32)
    x = jnp.maximum(x + b1[...], 0.0)
    x = (g1[...] * _BN_INV) * x + bt1[...]
    x = jnp.dot(w2[...].astype(bf), x.astype(bf),
                preferred_element_type=jnp.float32)
    x = jnp.maximum(x + b2[...], 0.0)
    x = (g2[...] * _BN_INV) * x + bt2[...]
    nrm = jnp.sqrt(jnp.sum(x * x, axis=0, keepdims=True))
    return x / jnp.maximum(nrm, 1e-12)


def _towers_kernel(*refs):
    eu = refs[0:4]
    ei = refs[4:8]
    wu = refs[8:20]
    wi = refs[20:32]
    ue_ref, ie_ref = refs[32], refs[33]
    ue_ref[...] = _tower_block(eu, wu)
    ie_ref[...] = _tower_block(ei, wi)


def _towers_tc(eu, ei, wu, wi):
    nblk = _B // _T_BLK
    e_spec = pl.BlockSpec((_EMB, _T_BLK), lambda i: (0, i))

    def _full(a):
        nd = a.ndim
        return pl.BlockSpec(a.shape, lambda i, _n=nd: (0,) * _n)

    in_specs = (
        [e_spec] * 8
        + [_full(a) for a in wu]
        + [_full(a) for a in wi]
    )
    out_spec = pl.BlockSpec((_HID[-1], _T_BLK), lambda i: (0, i))
    out_shape = (
        jax.ShapeDtypeStruct((_HID[-1], _B), jnp.float32),
        jax.ShapeDtypeStruct((_HID[-1], _B), jnp.float32),
    )
    return pl.pallas_call(
        _towers_kernel,
        grid=(nblk,),
        in_specs=in_specs,
        out_specs=(out_spec, out_spec),
        out_shape=out_shape,
    )(*eu, *ei, *wu, *wi)


_L_BLK = 512


def _logits_kernel(ue_ref, ie_ref, out_ref):
    out_ref[...] = lax.dot_general(
        ue_ref[...].astype(jnp.bfloat16),
        ie_ref[...].astype(jnp.bfloat16),
        (((0,), (0,)), ((), ())),
        preferred_element_type=jnp.float32,
    ) * (1.0 / _TEMP)


def _logits_tc(ue_t, ie_t):
    nblk = _B // _L_BLK
    return pl.pallas_call(
        _logits_kernel,
        grid=(nblk,),
        in_specs=[
            pl.BlockSpec((_HID[-1], _L_BLK), lambda i: (0, i)),
            pl.BlockSpec((_HID[-1], _B), lambda i: (0, 0)),
        ],
        out_specs=pl.BlockSpec((_L_BLK, _B), lambda i: (i, 0)),
        out_shape=jax.ShapeDtypeStruct((_B, _B), jnp.float32),
    )(ue_t, ie_t)


def kernel(
    user_id, emb_user_id,
    user_age, emb_user_age,
    user_gender, emb_user_gender,
    user_region, emb_user_region,
    item_id, emb_item_id,
    item_category, emb_item_category,
    item_brand, emb_item_brand,
    item_price_bucket, emb_item_price_bucket,
    u_W0, u_b0, u_g0, u_beta0,
    u_W1, u_b1, u_g1, u_beta1,
    u_W2, u_b2, u_g2, u_beta2,
    i_W0, i_b0, i_g0, i_beta0,
    i_W1, i_b1, i_g1, i_beta1,
    i_W2, i_b2, i_g2, i_beta2,
):
    idxs = [
        jnp.asarray(a, jnp.int32)
        for a in (user_id, user_age, user_gender, user_region,
                  item_id, item_category, item_brand, item_price_bucket)
    ]
    tabs_t = [t.T for t in (
        emb_user_id, emb_user_age, emb_user_gender, emb_user_region,
        emb_item_id, emb_item_category, emb_item_brand,
        emb_item_price_bucket)]
    tails_t = []
    for t in tabs_t:
        v = t.shape[1]
        nfull = v // _SEG
        tail = v - nfull * _SEG
        tl = t[:, nfull * _SEG:]
        if tail:
            tl = jnp.pad(tl, ((0, 0), (0, _SEG - tail)))
        else:
            tl = jnp.zeros((_EMB, _SEG), t.dtype)
        tails_t.append(tl)
    gathered = _sc_gather_t(idxs, tabs_t, tails_t)
    eu, ei = gathered[0:4], gathered[4:8]

    def _prep(w, b, g, bt):
        return (w.T, b.reshape(-1, 1), g.reshape(-1, 1), bt.reshape(-1, 1))

    wu = (*_prep(u_W0, u_b0, u_g0, u_beta0),
          *_prep(u_W1, u_b1, u_g1, u_beta1),
          *_prep(u_W2, u_b2, u_g2, u_beta2))
    wi = (*_prep(i_W0, i_b0, i_g0, i_beta0),
          *_prep(i_W1, i_b1, i_g1, i_beta1),
          *_prep(i_W2, i_b2, i_g2, i_beta2))

    ue_t, ie_t = _towers_tc(eu, ei, wu, wi)
    return _logits_tc(ue_t, ie_t)
